# Initial kernel scaffold; baseline (speedup 1.0000x reference)
import jax
import jax.numpy as jnp
from jax.experimental import pallas as pl


def kernel(x, edge_index_asd, edge_index_hc, batch, asd_W1, asd_b1, asd_a1, asd_W2, asd_b2, asd_a2, hc_W1, hc_b1, hc_a1, hc_W2, hc_b2, hc_a2, cls_W1, cls_b1, cls_a, cls_W2, cls_b2):
    raise NotImplementedError("write your pallas kernel here")


# speedup vs baseline: 8.7514x; 8.7514x over previous
import functools

import jax
import jax.numpy as jnp
from jax import lax
from jax.experimental import pallas as pl
from jax.experimental.pallas import tpu as pltpu
from jax.experimental.pallas import tpu_sc as plsc

N = 10000
E = 160000
NG = 64
NW = 32
CH = 125
NCH = (E // NW) // CH
DCH = (E // 16) // CH
DROW = 632
ND = 16 * DROW
NPT = 632
NPAD = 16 * NPT
BN = 1000

_MESH = plsc.VectorSubcoreMesh(core_axis_name="c", subcore_axis_name="s")


@functools.partial(
    pl.kernel,
    out_type=jax.ShapeDtypeStruct((2, NPAD, 128), jnp.float32),
    mesh=_MESH,
    scratch_types=[
        pltpu.VMEM((NCH, CH), jnp.int32),
        pltpu.VMEM((NCH, CH), jnp.int32),
        pltpu.VMEM((CH, 128), jnp.float32),
        pltpu.VMEM_SHARED((NPAD, 128), jnp.float32),
        pltpu.SemaphoreType.DMA,
    ],
)
def _prop_sc(g_hbm, rows_hbm, cols_hbm, zeros_hbm, out_hbm,
             rows_v, cols_v, buf, acc, sem):
    c = lax.axis_index("c")
    s = lax.axis_index("s")
    w = s * 2 + c
    pltpu.sync_copy(rows_hbm.at[w], rows_v)
    pltpu.sync_copy(cols_hbm.at[w], cols_v)
    base = s * NPT
    pltpu.sync_copy(zeros_hbm.at[pl.ds(base, NPT)], acc.at[pl.ds(base, NPT)])
    plsc.subcore_barrier()

    def step(j, carry):
        pltpu.async_copy(g_hbm.at[rows_v.at[j]], buf, sem).wait()
        pltpu.sync_copy(buf, acc.at[cols_v.at[j]], add=True)
        return carry

    lax.fori_loop(0, NCH, step, 0)
    plsc.subcore_barrier()
    pltpu.sync_copy(acc.at[pl.ds(base, NPT)], out_hbm.at[c, pl.ds(base, NPT)])


def _dis_body(degp_ref, dis_ref):
    deg = degp_ref[0] + degp_ref[1]
    dis = jnp.where(deg > 0, lax.rsqrt(jnp.maximum(deg, 1e-12)), 0.0)
    dis_ref[...] = dis[:, 0:1]


_dis_tc = pl.pallas_call(
    _dis_body,
    grid=(NPAD // BN + 1,),
    in_specs=[pl.BlockSpec((2, BN, 128), lambda i: (0, i, 0))],
    out_specs=pl.BlockSpec((BN, 1), lambda i: (i, 0)),
    out_shape=jax.ShapeDtypeStruct((NPAD, 1), jnp.float32),
)


def _pre_body(x_ref, w_ref, dis_ref, g1_ref, g2_ref, r_ref):
    p = jnp.dot(x_ref[...], w_ref[...], preferred_element_type=jnp.float32)
    dis = dis_ref[...]
    g1_ref[...] = dis * p[:, 256:384]
    g2_ref[...] = dis * p[:, 128:256]
    r_ref[...] = p[:, 0:128] - p[:, 256:384]


def _make_pre(cin):
    return pl.pallas_call(
        _pre_body,
        grid=(N // BN,),
        in_specs=[
            pl.BlockSpec((BN, cin), lambda i: (i, 0)),
            pl.BlockSpec((cin, 384), lambda i: (0, 0)),
            pl.BlockSpec((BN, 1), lambda i: (i, 0)),
        ],
        out_specs=[
            pl.BlockSpec((BN, 128), lambda i: (i, 0)),
            pl.BlockSpec((BN, 128), lambda i: (i, 0)),
            pl.BlockSpec((BN, 128), lambda i: (i, 0)),
        ],
        out_shape=[jax.ShapeDtypeStruct((N, 128), jnp.float32)] * 3,
    )


_pre_tc = _make_pre(256)


def _mid_body(g2_ref, ap_ref, dis_ref, q_ref):
    dis = dis_ref[...]
    q_ref[...] = g2_ref[...] - 2.0 * dis * dis * (ap_ref[0] + ap_ref[1])


_mid_tc = pl.pallas_call(
    _mid_body,
    grid=(N // BN,),
    in_specs=[
        pl.BlockSpec((BN, 128), lambda i: (i, 0)),
        pl.BlockSpec((2, BN, 128), lambda i: (0, i, 0)),
        pl.BlockSpec((BN, 1), lambda i: (i, 0)),
    ],
    out_specs=pl.BlockSpec((BN, 128), lambda i: (i, 0)),
    out_shape=jax.ShapeDtypeStruct((N, 128), jnp.float32),
)


def _postpre_body(r_ref, ap_ref, dis_ref, b_ref, al_ref, w_ref,
                  g1_ref, g2_ref, r2_ref):
    dis = dis_ref[...]
    out1 = r_ref[...] - dis * (ap_ref[0] + ap_ref[1]) + b_ref[...]
    x2 = jnp.where(out1 >= 0, out1, al_ref[...] * out1)
    p = jnp.dot(x2, w_ref[...], preferred_element_type=jnp.float32)
    g1_ref[...] = dis * p[:, 256:384]
    g2_ref[...] = dis * p[:, 128:256]
    r2_ref[...] = p[:, 0:128] - p[:, 256:384]


_postpre_tc = pl.pallas_call(
    _postpre_body,
    grid=(N // BN,),
    in_specs=[
        pl.BlockSpec((BN, 128), lambda i: (i, 0)),
        pl.BlockSpec((2, BN, 128), lambda i: (0, i, 0)),
        pl.BlockSpec((BN, 1), lambda i: (i, 0)),
        pl.BlockSpec((1, 128), lambda i: (0, 0)),
        pl.BlockSpec((1, 128), lambda i: (0, 0)),
        pl.BlockSpec((128, 384), lambda i: (0, 0)),
    ],
    out_specs=[
        pl.BlockSpec((BN, 128), lambda i: (i, 0)),
        pl.BlockSpec((BN, 128), lambda i: (i, 0)),
        pl.BlockSpec((BN, 128), lambda i: (i, 0)),
    ],
    out_shape=[jax.ShapeDtypeStruct((N, 128), jnp.float32)] * 3,
)


def _pool_body(r_ref, ap_ref, dis_ref, b_ref, al_ref, batch_ref,
               z_ref, zsum, csum):
    i = pl.program_id(0)

    @pl.when(i == 0)
    def _init():
        zsum[...] = jnp.zeros_like(zsum)
        csum[...] = jnp.zeros_like(csum)

    dis = dis_ref[...]
    out2 = r_ref[...] - dis * (ap_ref[0] + ap_ref[1]) + b_ref[...]
    h = jnp.where(out2 >= 0, out2, al_ref[...] * out2)
    gids = lax.broadcasted_iota(jnp.int32, (BN, NG), 1)
    mask = (batch_ref[...] == gids).astype(jnp.float32)
    dn = (((0,), (0,)), ((), ()))
    zsum[...] += lax.dot_general(mask, h, dn,
                                 preferred_element_type=jnp.float32)
    csum[...] += lax.dot_general(mask, jnp.ones_like(h), dn,
                                 preferred_element_type=jnp.float32)

    @pl.when(i == N // BN - 1)
    def _fin():
        z_ref[...] = zsum[...] / jnp.maximum(csum[...], 1.0)


_pool_tc = pl.pallas_call(
    _pool_body,
    grid=(N // BN,),
    in_specs=[
        pl.BlockSpec((BN, 128), lambda i: (i, 0)),
        pl.BlockSpec((2, BN, 128), lambda i: (0, i, 0)),
        pl.BlockSpec((BN, 1), lambda i: (i, 0)),
        pl.BlockSpec((1, 128), lambda i: (0, 0)),
        pl.BlockSpec((1, 128), lambda i: (0, 0)),
        pl.BlockSpec((BN, 1), lambda i: (i, 0)),
    ],
    out_specs=pl.BlockSpec((NG, 128), lambda i: (0, 0)),
    out_shape=jax.ShapeDtypeStruct((NG, 128), jnp.float32),
    scratch_shapes=[
        pltpu.VMEM((NG, 128), jnp.float32),
        pltpu.VMEM((NG, 128), jnp.float32),
    ],
)


def _cls_body(za_ref, zh_ref, w1_ref, b1_ref, a_ref, w2_ref, b2_ref,
              logits_ref, z_ref):
    z = jnp.concatenate([za_ref[...], zh_ref[...]], axis=1)
    h0 = jnp.dot(z, w1_ref[...], preferred_element_type=jnp.float32) + b1_ref[...]
    h = jnp.where(h0 >= 0, h0, a_ref[...] * h0)
    logits_ref[...] = (jnp.dot(h, w2_ref[...],
                               preferred_element_type=jnp.float32)
                       + b2_ref[...])
    z_ref[...] = z


_cls_tc = pl.pallas_call(
    _cls_body,
    out_shape=[
        jax.ShapeDtypeStruct((NG, 2), jnp.float32),
        jax.ShapeDtypeStruct((NG, 256), jnp.float32),
    ],
)


def kernel(x, edge_index_asd, edge_index_hc, batch,
           asd_W1, asd_b1, asd_a1, asd_W2, asd_b2, asd_a2,
           hc_W1, hc_b1, hc_a1, hc_W2, hc_b2, hc_a2,
           cls_W1, cls_b1, cls_a, cls_W2, cls_b2):
    zeros128 = jnp.zeros((NPAD, 128), jnp.float32)
    ones_nodes = jnp.ones((N, 128), jnp.float32)

    rows_a = edge_index_asd[0].reshape(NW, NCH, CH)
    rows_h = edge_index_hc[0].reshape(NW, NCH, CH)
    degp_a = _prop_sc(ones_nodes, rows_a, rows_a, zeros128)
    degp_h = _prop_sc(ones_nodes, rows_h, rows_h, zeros128)
    dis_a = _dis_tc(degp_a)[:N]
    dis_h = _dis_tc(degp_h)[:N]
    batch2 = batch.reshape(N, 1)

    def run_branch(ei, dis, W1, b1, a1, W2, b2, a2):
        rows = ei[0].reshape(NW, NCH, CH)
        cols = ei[1].reshape(NW, NCH, CH)
        wc1 = jnp.concatenate([W1[0], W1[1], W1[2]], axis=1)
        wc2 = jnp.concatenate([W2[0], W2[1], W2[2]], axis=1)
        g1, g2, r = _pre_tc(x, wc1, dis)
        ap1 = _prop_sc(g1, rows, cols, zeros128)
        q = _mid_tc(g2, ap1, dis)
        ap2 = _prop_sc(q, rows, cols, zeros128)
        g1b, g2b, rb = _postpre_tc(r, ap2, dis, b1.reshape(1, 128),
                                   a1.reshape(1, 128), wc2)
        ap3 = _prop_sc(g1b, rows, cols, zeros128)
        qb = _mid_tc(g2b, ap3, dis)
        ap4 = _prop_sc(qb, rows, cols, zeros128)
        return _pool_tc(rb, ap4, dis, b2.reshape(1, 128),
                        a2.reshape(1, 128), batch2)

    z_a = run_branch(edge_index_asd, dis_a,
                     asd_W1, asd_b1, asd_a1, asd_W2, asd_b2, asd_a2)
    z_h = run_branch(edge_index_hc, dis_h,
                     hc_W1, hc_b1, hc_a1, hc_W2, hc_b2, hc_a2)
    logits, z = _cls_tc(z_a, z_h, cls_W1, cls_b1.reshape(1, 256),
                        cls_a.reshape(1, 256), cls_W2, cls_b2.reshape(1, 2))
    return logits, z


# speedup vs baseline: 12.7485x; 1.4567x over previous
import functools

import jax
import jax.numpy as jnp
from jax import lax
from jax.experimental import pallas as pl
from jax.experimental.pallas import tpu as pltpu
from jax.experimental.pallas import tpu_sc as plsc

N = 10000
E = 160000
NG = 64
NW = 32
CH = 125
NCH = (E // NW) // CH
DCH = (E // 16) // CH
DROW = 632
ND = 16 * DROW
NPT = 632
NPAD = 16 * NPT
BN = 1000

_MESH = plsc.VectorSubcoreMesh(core_axis_name="c", subcore_axis_name="s")


@functools.partial(
    pl.kernel,
    out_type=jax.ShapeDtypeStruct((2, NPAD, 128), jnp.float32),
    mesh=_MESH,
    scratch_types=[
        pltpu.VMEM((NCH, CH), jnp.int32),
        pltpu.VMEM((NCH, CH), jnp.int32),
        pltpu.VMEM((CH, 128), jnp.float32),
        pltpu.VMEM((CH, 128), jnp.float32),
        pltpu.VMEM_SHARED((NPAD, 128), jnp.float32),
        pltpu.SemaphoreType.DMA,
        pltpu.SemaphoreType.DMA,
    ],
)
def _prop_sc(g_hbm, rows_hbm, cols_hbm, zeros_hbm, out_hbm,
             rows_v, cols_v, buf0, buf1, acc, sem0, sem1):
    c = lax.axis_index("c")
    s = lax.axis_index("s")
    w = s * 2 + c
    pltpu.sync_copy(rows_hbm.at[w], rows_v)
    pltpu.sync_copy(cols_hbm.at[w], cols_v)
    base = s * NPT
    pltpu.sync_copy(zeros_hbm.at[pl.ds(base, NPT)], acc.at[pl.ds(base, NPT)])
    plsc.subcore_barrier()

    pltpu.async_copy(g_hbm.at[rows_v.at[0]], buf0, sem0)
    pltpu.async_copy(g_hbm.at[rows_v.at[1]], buf1, sem1)

    def step(i, carry):
        j0 = 2 * i
        pltpu.make_async_copy(g_hbm.at[rows_v.at[j0]], buf0, sem0).wait()
        pltpu.sync_copy(buf0, acc.at[cols_v.at[j0]], add=True)
        pltpu.async_copy(g_hbm.at[rows_v.at[j0 + 2]], buf0, sem0)
        pltpu.make_async_copy(g_hbm.at[rows_v.at[j0 + 1]], buf1, sem1).wait()
        pltpu.sync_copy(buf1, acc.at[cols_v.at[j0 + 1]], add=True)
        pltpu.async_copy(g_hbm.at[rows_v.at[j0 + 3]], buf1, sem1)
        return carry

    lax.fori_loop(0, NCH // 2 - 1, step, 0)
    j0 = NCH - 2
    pltpu.make_async_copy(g_hbm.at[rows_v.at[j0]], buf0, sem0).wait()
    pltpu.sync_copy(buf0, acc.at[cols_v.at[j0]], add=True)
    pltpu.make_async_copy(g_hbm.at[rows_v.at[j0 + 1]], buf1, sem1).wait()
    pltpu.sync_copy(buf1, acc.at[cols_v.at[j0 + 1]], add=True)
    plsc.subcore_barrier()
    pltpu.sync_copy(acc.at[pl.ds(base, NPT)], out_hbm.at[c, pl.ds(base, NPT)])


def _dis_body(degp_ref, dis_ref):
    deg = degp_ref[0] + degp_ref[1]
    dis = jnp.where(deg > 0, lax.rsqrt(jnp.maximum(deg, 1e-12)), 0.0)
    dis_ref[...] = dis[:, 0:1]


_dis_tc = pl.pallas_call(
    _dis_body,
    grid=(NPAD // BN + 1,),
    in_specs=[pl.BlockSpec((2, BN, 128), lambda i: (0, i, 0))],
    out_specs=pl.BlockSpec((BN, 1), lambda i: (i, 0)),
    out_shape=jax.ShapeDtypeStruct((NPAD, 1), jnp.float32),
)


def _pre_body(x_ref, w_ref, dis_ref, g1_ref, g2_ref, r_ref):
    p = jnp.dot(x_ref[...], w_ref[...], preferred_element_type=jnp.float32)
    dis = dis_ref[...]
    g1_ref[...] = dis * p[:, 256:384]
    g2_ref[...] = dis * p[:, 128:256]
    r_ref[...] = p[:, 0:128] - p[:, 256:384]


def _make_pre(cin):
    return pl.pallas_call(
        _pre_body,
        grid=(N // BN,),
        in_specs=[
            pl.BlockSpec((BN, cin), lambda i: (i, 0)),
            pl.BlockSpec((cin, 384), lambda i: (0, 0)),
            pl.BlockSpec((BN, 1), lambda i: (i, 0)),
        ],
        out_specs=[
            pl.BlockSpec((BN, 128), lambda i: (i, 0)),
            pl.BlockSpec((BN, 128), lambda i: (i, 0)),
            pl.BlockSpec((BN, 128), lambda i: (i, 0)),
        ],
        out_shape=[jax.ShapeDtypeStruct((N, 128), jnp.float32)] * 3,
    )


_pre_tc = _make_pre(256)


def _mid_body(g2_ref, ap_ref, dis_ref, q_ref):
    dis = dis_ref[...]
    q_ref[...] = g2_ref[...] - 2.0 * dis * dis * (ap_ref[0] + ap_ref[1])


_mid_tc = pl.pallas_call(
    _mid_body,
    grid=(N // BN,),
    in_specs=[
        pl.BlockSpec((BN, 128), lambda i: (i, 0)),
        pl.BlockSpec((2, BN, 128), lambda i: (0, i, 0)),
        pl.BlockSpec((BN, 1), lambda i: (i, 0)),
    ],
    out_specs=pl.BlockSpec((BN, 128), lambda i: (i, 0)),
    out_shape=jax.ShapeDtypeStruct((N, 128), jnp.float32),
)


def _postpre_body(r_ref, ap_ref, dis_ref, b_ref, al_ref, w_ref,
                  g1_ref, g2_ref, r2_ref):
    dis = dis_ref[...]
    out1 = r_ref[...] - dis * (ap_ref[0] + ap_ref[1]) + b_ref[...]
    x2 = jnp.where(out1 >= 0, out1, al_ref[...] * out1)
    p = jnp.dot(x2, w_ref[...], preferred_element_type=jnp.float32)
    g1_ref[...] = dis * p[:, 256:384]
    g2_ref[...] = dis * p[:, 128:256]
    r2_ref[...] = p[:, 0:128] - p[:, 256:384]


_postpre_tc = pl.pallas_call(
    _postpre_body,
    grid=(N // BN,),
    in_specs=[
        pl.BlockSpec((BN, 128), lambda i: (i, 0)),
        pl.BlockSpec((2, BN, 128), lambda i: (0, i, 0)),
        pl.BlockSpec((BN, 1), lambda i: (i, 0)),
        pl.BlockSpec((1, 128), lambda i: (0, 0)),
        pl.BlockSpec((1, 128), lambda i: (0, 0)),
        pl.BlockSpec((128, 384), lambda i: (0, 0)),
    ],
    out_specs=[
        pl.BlockSpec((BN, 128), lambda i: (i, 0)),
        pl.BlockSpec((BN, 128), lambda i: (i, 0)),
        pl.BlockSpec((BN, 128), lambda i: (i, 0)),
    ],
    out_shape=[jax.ShapeDtypeStruct((N, 128), jnp.float32)] * 3,
)


def _pool_body(r_ref, ap_ref, dis_ref, b_ref, al_ref, batch_ref,
               z_ref, zsum, csum):
    i = pl.program_id(0)

    @pl.when(i == 0)
    def _init():
        zsum[...] = jnp.zeros_like(zsum)
        csum[...] = jnp.zeros_like(csum)

    dis = dis_ref[...]
    out2 = r_ref[...] - dis * (ap_ref[0] + ap_ref[1]) + b_ref[...]
    h = jnp.where(out2 >= 0, out2, al_ref[...] * out2)
    gids = lax.broadcasted_iota(jnp.int32, (BN, NG), 1)
    mask = (batch_ref[...] == gids).astype(jnp.float32)
    dn = (((0,), (0,)), ((), ()))
    zsum[...] += lax.dot_general(mask, h, dn,
                                 preferred_element_type=jnp.float32)
    csum[...] += lax.dot_general(mask, jnp.ones_like(h), dn,
                                 preferred_element_type=jnp.float32)

    @pl.when(i == N // BN - 1)
    def _fin():
        z_ref[...] = zsum[...] / jnp.maximum(csum[...], 1.0)


_pool_tc = pl.pallas_call(
    _pool_body,
    grid=(N // BN,),
    in_specs=[
        pl.BlockSpec((BN, 128), lambda i: (i, 0)),
        pl.BlockSpec((2, BN, 128), lambda i: (0, i, 0)),
        pl.BlockSpec((BN, 1), lambda i: (i, 0)),
        pl.BlockSpec((1, 128), lambda i: (0, 0)),
        pl.BlockSpec((1, 128), lambda i: (0, 0)),
        pl.BlockSpec((BN, 1), lambda i: (i, 0)),
    ],
    out_specs=pl.BlockSpec((NG, 128), lambda i: (0, 0)),
    out_shape=jax.ShapeDtypeStruct((NG, 128), jnp.float32),
    scratch_shapes=[
        pltpu.VMEM((NG, 128), jnp.float32),
        pltpu.VMEM((NG, 128), jnp.float32),
    ],
)


def _cls_body(za_ref, zh_ref, w1_ref, b1_ref, a_ref, w2_ref, b2_ref,
              logits_ref, z_ref):
    z = jnp.concatenate([za_ref[...], zh_ref[...]], axis=1)
    h0 = jnp.dot(z, w1_ref[...], preferred_element_type=jnp.float32) + b1_ref[...]
    h = jnp.where(h0 >= 0, h0, a_ref[...] * h0)
    logits_ref[...] = (jnp.dot(h, w2_ref[...],
                               preferred_element_type=jnp.float32)
                       + b2_ref[...])
    z_ref[...] = z


_cls_tc = pl.pallas_call(
    _cls_body,
    out_shape=[
        jax.ShapeDtypeStruct((NG, 2), jnp.float32),
        jax.ShapeDtypeStruct((NG, 256), jnp.float32),
    ],
)


def kernel(x, edge_index_asd, edge_index_hc, batch,
           asd_W1, asd_b1, asd_a1, asd_W2, asd_b2, asd_a2,
           hc_W1, hc_b1, hc_a1, hc_W2, hc_b2, hc_a2,
           cls_W1, cls_b1, cls_a, cls_W2, cls_b2):
    zeros128 = jnp.zeros((NPAD, 128), jnp.float32)
    ones_nodes = jnp.ones((N, 128), jnp.float32)

    rows_a = edge_index_asd[0].reshape(NW, NCH, CH)
    rows_h = edge_index_hc[0].reshape(NW, NCH, CH)
    degp_a = _prop_sc(ones_nodes, rows_a, rows_a, zeros128)
    degp_h = _prop_sc(ones_nodes, rows_h, rows_h, zeros128)
    dis_a = _dis_tc(degp_a)[:N]
    dis_h = _dis_tc(degp_h)[:N]
    batch2 = batch.reshape(N, 1)

    def run_branch(ei, dis, W1, b1, a1, W2, b2, a2):
        rows = ei[0].reshape(NW, NCH, CH)
        cols = ei[1].reshape(NW, NCH, CH)
        wc1 = jnp.concatenate([W1[0], W1[1], W1[2]], axis=1)
        wc2 = jnp.concatenate([W2[0], W2[1], W2[2]], axis=1)
        g1, g2, r = _pre_tc(x, wc1, dis)
        ap1 = _prop_sc(g1, rows, cols, zeros128)
        q = _mid_tc(g2, ap1, dis)
        ap2 = _prop_sc(q, rows, cols, zeros128)
        g1b, g2b, rb = _postpre_tc(r, ap2, dis, b1.reshape(1, 128),
                                   a1.reshape(1, 128), wc2)
        ap3 = _prop_sc(g1b, rows, cols, zeros128)
        qb = _mid_tc(g2b, ap3, dis)
        ap4 = _prop_sc(qb, rows, cols, zeros128)
        return _pool_tc(rb, ap4, dis, b2.reshape(1, 128),
                        a2.reshape(1, 128), batch2)

    z_a = run_branch(edge_index_asd, dis_a,
                     asd_W1, asd_b1, asd_a1, asd_W2, asd_b2, asd_a2)
    z_h = run_branch(edge_index_hc, dis_h,
                     hc_W1, hc_b1, hc_a1, hc_W2, hc_b2, hc_a2)
    logits, z = _cls_tc(z_a, z_h, cls_W1, cls_b1.reshape(1, 256),
                        cls_a.reshape(1, 256), cls_W2, cls_b2.reshape(1, 2))
    return logits, z


# speedup vs baseline: 13.4251x; 1.0531x over previous
import functools

import jax
import jax.numpy as jnp
from jax import lax
from jax.experimental import pallas as pl
from jax.experimental.pallas import tpu as pltpu
from jax.experimental.pallas import tpu_sc as plsc

N = 10000
E = 160000
NG = 64
NW = 32
CH = 125
NCH = (E // NW) // CH
DCH = (E // 16) // CH
DROW = 632
ND = 16 * DROW
NPT = 632
NPAD = 16 * NPT
BN = 1000

_MESH = plsc.VectorSubcoreMesh(core_axis_name="c", subcore_axis_name="s")


@functools.partial(
    pl.kernel,
    out_type=jax.ShapeDtypeStruct((2, NPAD, 128), jnp.float32),
    mesh=_MESH,
    scratch_types=[
        pltpu.VMEM((NCH, CH), jnp.int32),
        pltpu.VMEM((NCH, CH), jnp.int32),
        pltpu.VMEM((CH, 128), jnp.float32),
        pltpu.VMEM((CH, 128), jnp.float32),
        pltpu.VMEM_SHARED((NPAD, 128), jnp.float32),
        pltpu.SemaphoreType.DMA,
        pltpu.SemaphoreType.DMA,
    ],
)
def _prop_sc(g_hbm, rows_hbm, cols_hbm, zeros_hbm, out_hbm,
             rows_v, cols_v, buf0, buf1, acc, sem0, sem1):
    c = lax.axis_index("c")
    s = lax.axis_index("s")
    w = s * 2 + c
    pltpu.sync_copy(rows_hbm.at[w], rows_v)
    pltpu.sync_copy(cols_hbm.at[w], cols_v)
    base = s * NPT
    pltpu.sync_copy(zeros_hbm.at[pl.ds(base, NPT)], acc.at[pl.ds(base, NPT)])
    plsc.subcore_barrier()

    pltpu.async_copy(g_hbm.at[rows_v.at[0]], buf0, sem0)
    pltpu.async_copy(g_hbm.at[rows_v.at[1]], buf1, sem1)

    def step(i, carry):
        j0 = 2 * i
        pltpu.make_async_copy(g_hbm.at[rows_v.at[j0]], buf0, sem0).wait()
        pltpu.sync_copy(buf0, acc.at[cols_v.at[j0]], add=True)
        pltpu.async_copy(g_hbm.at[rows_v.at[j0 + 2]], buf0, sem0)
        pltpu.make_async_copy(g_hbm.at[rows_v.at[j0 + 1]], buf1, sem1).wait()
        pltpu.sync_copy(buf1, acc.at[cols_v.at[j0 + 1]], add=True)
        pltpu.async_copy(g_hbm.at[rows_v.at[j0 + 3]], buf1, sem1)
        return carry

    lax.fori_loop(0, NCH // 2 - 1, step, 0)
    j0 = NCH - 2
    pltpu.make_async_copy(g_hbm.at[rows_v.at[j0]], buf0, sem0).wait()
    pltpu.sync_copy(buf0, acc.at[cols_v.at[j0]], add=True)
    pltpu.make_async_copy(g_hbm.at[rows_v.at[j0 + 1]], buf1, sem1).wait()
    pltpu.sync_copy(buf1, acc.at[cols_v.at[j0 + 1]], add=True)
    plsc.subcore_barrier()
    pltpu.sync_copy(acc.at[pl.ds(base, NPT)], out_hbm.at[c, pl.ds(base, NPT)])


@functools.partial(
    pl.kernel,
    out_type=jax.ShapeDtypeStruct((2, NPAD, 128), jnp.float32),
    mesh=_MESH,
    scratch_types=[
        pltpu.VMEM((NCH, CH), jnp.int32),
        pltpu.VMEM((CH, 128), jnp.float32),
        pltpu.VMEM_SHARED((NPAD, 128), jnp.float32),
    ],
)
def _deg_sc(rows_hbm, ones_hbm, zeros_hbm, out_hbm, idx_v, ones_v, acc):
    c = lax.axis_index("c")
    s = lax.axis_index("s")
    w = s * 2 + c
    base = s * NPT
    pltpu.sync_copy(ones_hbm, ones_v)
    pltpu.sync_copy(zeros_hbm.at[pl.ds(base, NPT)], acc.at[pl.ds(base, NPT)])
    plsc.subcore_barrier()
    pltpu.sync_copy(rows_hbm.at[w], idx_v)

    def step(j, carry):
        pltpu.sync_copy(ones_v, acc.at[idx_v.at[j]], add=True)
        return carry

    lax.fori_loop(0, NCH, step, 0)
    plsc.subcore_barrier()
    pltpu.sync_copy(acc.at[pl.ds(base, NPT)], out_hbm.at[c, pl.ds(base, NPT)])


def _dis_body(degp_ref, dis_ref):
    deg = degp_ref[0] + degp_ref[1]
    dis = jnp.where(deg > 0, lax.rsqrt(jnp.maximum(deg, 1e-12)), 0.0)
    dis_ref[...] = dis[:, 0:1]


_dis_tc = pl.pallas_call(
    _dis_body,
    grid=(NPAD // BN + 1,),
    in_specs=[pl.BlockSpec((2, BN, 128), lambda i: (0, i, 0))],
    out_specs=pl.BlockSpec((BN, 1), lambda i: (i, 0)),
    out_shape=jax.ShapeDtypeStruct((NPAD, 1), jnp.float32),
)


def _pre_body(x_ref, w_ref, dis_ref, g1_ref, g2_ref, r_ref):
    p = jnp.dot(x_ref[...], w_ref[...], preferred_element_type=jnp.float32)
    dis = dis_ref[...]
    g1_ref[...] = dis * p[:, 256:384]
    g2_ref[...] = dis * p[:, 128:256]
    r_ref[...] = p[:, 0:128] - p[:, 256:384]


def _make_pre(cin):
    return pl.pallas_call(
        _pre_body,
        grid=(N // BN,),
        in_specs=[
            pl.BlockSpec((BN, cin), lambda i: (i, 0)),
            pl.BlockSpec((cin, 384), lambda i: (0, 0)),
            pl.BlockSpec((BN, 1), lambda i: (i, 0)),
        ],
        out_specs=[
            pl.BlockSpec((BN, 128), lambda i: (i, 0)),
            pl.BlockSpec((BN, 128), lambda i: (i, 0)),
            pl.BlockSpec((BN, 128), lambda i: (i, 0)),
        ],
        out_shape=[jax.ShapeDtypeStruct((N, 128), jnp.float32)] * 3,
    )


_pre_tc = _make_pre(256)


def _mid_body(g2_ref, ap_ref, dis_ref, q_ref):
    dis = dis_ref[...]
    q_ref[...] = g2_ref[...] - 2.0 * dis * dis * (ap_ref[0] + ap_ref[1])


_mid_tc = pl.pallas_call(
    _mid_body,
    grid=(N // BN,),
    in_specs=[
        pl.BlockSpec((BN, 128), lambda i: (i, 0)),
        pl.BlockSpec((2, BN, 128), lambda i: (0, i, 0)),
        pl.BlockSpec((BN, 1), lambda i: (i, 0)),
    ],
    out_specs=pl.BlockSpec((BN, 128), lambda i: (i, 0)),
    out_shape=jax.ShapeDtypeStruct((N, 128), jnp.float32),
)


def _postpre_body(r_ref, ap_ref, dis_ref, b_ref, al_ref, w_ref,
                  g1_ref, g2_ref, r2_ref):
    dis = dis_ref[...]
    out1 = r_ref[...] - dis * (ap_ref[0] + ap_ref[1]) + b_ref[...]
    x2 = jnp.where(out1 >= 0, out1, al_ref[...] * out1)
    p = jnp.dot(x2, w_ref[...], preferred_element_type=jnp.float32)
    g1_ref[...] = dis * p[:, 256:384]
    g2_ref[...] = dis * p[:, 128:256]
    r2_ref[...] = p[:, 0:128] - p[:, 256:384]


_postpre_tc = pl.pallas_call(
    _postpre_body,
    grid=(N // BN,),
    in_specs=[
        pl.BlockSpec((BN, 128), lambda i: (i, 0)),
        pl.BlockSpec((2, BN, 128), lambda i: (0, i, 0)),
        pl.BlockSpec((BN, 1), lambda i: (i, 0)),
        pl.BlockSpec((1, 128), lambda i: (0, 0)),
        pl.BlockSpec((1, 128), lambda i: (0, 0)),
        pl.BlockSpec((128, 384), lambda i: (0, 0)),
    ],
    out_specs=[
        pl.BlockSpec((BN, 128), lambda i: (i, 0)),
        pl.BlockSpec((BN, 128), lambda i: (i, 0)),
        pl.BlockSpec((BN, 128), lambda i: (i, 0)),
    ],
    out_shape=[jax.ShapeDtypeStruct((N, 128), jnp.float32)] * 3,
)


def _pool_body(r_ref, ap_ref, dis_ref, b_ref, al_ref, batch_ref,
               z_ref, zsum, csum):
    i = pl.program_id(0)

    @pl.when(i == 0)
    def _init():
        zsum[...] = jnp.zeros_like(zsum)
        csum[...] = jnp.zeros_like(csum)

    dis = dis_ref[...]
    out2 = r_ref[...] - dis * (ap_ref[0] + ap_ref[1]) + b_ref[...]
    h = jnp.where(out2 >= 0, out2, al_ref[...] * out2)
    gids = lax.broadcasted_iota(jnp.int32, (BN, NG), 1)
    mask = (batch_ref[...] == gids).astype(jnp.float32)
    dn = (((0,), (0,)), ((), ()))
    zsum[...] += lax.dot_general(mask, h, dn,
                                 preferred_element_type=jnp.float32)
    csum[...] += lax.dot_general(mask, jnp.ones_like(h), dn,
                                 preferred_element_type=jnp.float32)

    @pl.when(i == N // BN - 1)
    def _fin():
        z_ref[...] = zsum[...] / jnp.maximum(csum[...], 1.0)


_pool_tc = pl.pallas_call(
    _pool_body,
    grid=(N // BN,),
    in_specs=[
        pl.BlockSpec((BN, 128), lambda i: (i, 0)),
        pl.BlockSpec((2, BN, 128), lambda i: (0, i, 0)),
        pl.BlockSpec((BN, 1), lambda i: (i, 0)),
        pl.BlockSpec((1, 128), lambda i: (0, 0)),
        pl.BlockSpec((1, 128), lambda i: (0, 0)),
        pl.BlockSpec((BN, 1), lambda i: (i, 0)),
    ],
    out_specs=pl.BlockSpec((NG, 128), lambda i: (0, 0)),
    out_shape=jax.ShapeDtypeStruct((NG, 128), jnp.float32),
    scratch_shapes=[
        pltpu.VMEM((NG, 128), jnp.float32),
        pltpu.VMEM((NG, 128), jnp.float32),
    ],
)


def _cls_body(za_ref, zh_ref, w1_ref, b1_ref, a_ref, w2_ref, b2_ref,
              logits_ref, z_ref):
    z = jnp.concatenate([za_ref[...], zh_ref[...]], axis=1)
    h0 = jnp.dot(z, w1_ref[...], preferred_element_type=jnp.float32) + b1_ref[...]
    h = jnp.where(h0 >= 0, h0, a_ref[...] * h0)
    logits_ref[...] = (jnp.dot(h, w2_ref[...],
                               preferred_element_type=jnp.float32)
                       + b2_ref[...])
    z_ref[...] = z


_cls_tc = pl.pallas_call(
    _cls_body,
    out_shape=[
        jax.ShapeDtypeStruct((NG, 2), jnp.float32),
        jax.ShapeDtypeStruct((NG, 256), jnp.float32),
    ],
)


def kernel(x, edge_index_asd, edge_index_hc, batch,
           asd_W1, asd_b1, asd_a1, asd_W2, asd_b2, asd_a2,
           hc_W1, hc_b1, hc_a1, hc_W2, hc_b2, hc_a2,
           cls_W1, cls_b1, cls_a, cls_W2, cls_b2):
    zeros128 = jnp.zeros((NPAD, 128), jnp.float32)
    ones128 = jnp.ones((CH, 128), jnp.float32)

    rows_a = edge_index_asd[0].reshape(NW, NCH, CH)
    rows_h = edge_index_hc[0].reshape(NW, NCH, CH)
    degp_a = _deg_sc(rows_a, ones128, zeros128)
    degp_h = _deg_sc(rows_h, ones128, zeros128)
    dis_a = _dis_tc(degp_a)[:N]
    dis_h = _dis_tc(degp_h)[:N]
    batch2 = batch.reshape(N, 1)

    def run_branch(ei, dis, W1, b1, a1, W2, b2, a2):
        rows = ei[0].reshape(NW, NCH, CH)
        cols = ei[1].reshape(NW, NCH, CH)
        wc1 = jnp.concatenate([W1[0], W1[1], W1[2]], axis=1)
        wc2 = jnp.concatenate([W2[0], W2[1], W2[2]], axis=1)
        g1, g2, r = _pre_tc(x, wc1, dis)
        ap1 = _prop_sc(g1, rows, cols, zeros128)
        q = _mid_tc(g2, ap1, dis)
        ap2 = _prop_sc(q, rows, cols, zeros128)
        g1b, g2b, rb = _postpre_tc(r, ap2, dis, b1.reshape(1, 128),
                                   a1.reshape(1, 128), wc2)
        ap3 = _prop_sc(g1b, rows, cols, zeros128)
        qb = _mid_tc(g2b, ap3, dis)
        ap4 = _prop_sc(qb, rows, cols, zeros128)
        return _pool_tc(rb, ap4, dis, b2.reshape(1, 128),
                        a2.reshape(1, 128), batch2)

    z_a = run_branch(edge_index_asd, dis_a,
                     asd_W1, asd_b1, asd_a1, asd_W2, asd_b2, asd_a2)
    z_h = run_branch(edge_index_hc, dis_h,
                     hc_W1, hc_b1, hc_a1, hc_W2, hc_b2, hc_a2)
    logits, z = _cls_tc(z_a, z_h, cls_W1, cls_b1.reshape(1, 256),
                        cls_a.reshape(1, 256), cls_W2, cls_b2.reshape(1, 2))
    return logits, z


# speedup vs baseline: 13.4666x; 1.0031x over previous
import functools

import jax
import jax.numpy as jnp
from jax import lax
from jax.experimental import pallas as pl
from jax.experimental.pallas import tpu as pltpu
from jax.experimental.pallas import tpu_sc as plsc

N = 10000
E = 160000
NG = 64
NW = 32
CH = 125
NCH = (E // NW) // CH
NPT = 632
NPAD = 16 * NPT
BN = 1000

_MESH = plsc.VectorSubcoreMesh(core_axis_name="c", subcore_axis_name="s")


@functools.partial(
    pl.kernel,
    out_type=jax.ShapeDtypeStruct((2, NPAD, 128), jnp.float32),
    mesh=_MESH,
    scratch_types=[
        pltpu.VMEM((NCH, CH), jnp.int32),
        pltpu.VMEM((NCH, CH), jnp.int32),
        pltpu.VMEM((CH, 128), jnp.float32),
        pltpu.VMEM((CH, 128), jnp.float32),
        pltpu.VMEM_SHARED((NPAD, 128), jnp.float32),
        pltpu.SemaphoreType.DMA,
        pltpu.SemaphoreType.DMA,
    ],
)
def _prop_sc(g_hbm, rows_hbm, cols_hbm, zeros_hbm, out_hbm,
             rows_v, cols_v, buf0, buf1, acc, sem0, sem1):
    c = lax.axis_index("c")
    s = lax.axis_index("s")
    w = s * 2 + c
    pltpu.sync_copy(rows_hbm.at[w], rows_v)
    pltpu.sync_copy(cols_hbm.at[w], cols_v)
    base = s * NPT
    pltpu.sync_copy(zeros_hbm.at[pl.ds(base, NPT)], acc.at[pl.ds(base, NPT)])
    plsc.subcore_barrier()

    pltpu.async_copy(g_hbm.at[rows_v.at[0]], buf0, sem0)
    pltpu.async_copy(g_hbm.at[rows_v.at[1]], buf1, sem1)

    def step(i, carry):
        j0 = 2 * i
        pltpu.make_async_copy(g_hbm.at[rows_v.at[j0]], buf0, sem0).wait()
        pltpu.sync_copy(buf0, acc.at[cols_v.at[j0]], add=True)
        pltpu.async_copy(g_hbm.at[rows_v.at[j0 + 2]], buf0, sem0)
        pltpu.make_async_copy(g_hbm.at[rows_v.at[j0 + 1]], buf1, sem1).wait()
        pltpu.sync_copy(buf1, acc.at[cols_v.at[j0 + 1]], add=True)
        pltpu.async_copy(g_hbm.at[rows_v.at[j0 + 3]], buf1, sem1)
        return carry

    lax.fori_loop(0, NCH // 2 - 1, step, 0)
    j0 = NCH - 2
    pltpu.make_async_copy(g_hbm.at[rows_v.at[j0]], buf0, sem0).wait()
    pltpu.sync_copy(buf0, acc.at[cols_v.at[j0]], add=True)
    pltpu.make_async_copy(g_hbm.at[rows_v.at[j0 + 1]], buf1, sem1).wait()
    pltpu.sync_copy(buf1, acc.at[cols_v.at[j0 + 1]], add=True)
    plsc.subcore_barrier()
    pltpu.sync_copy(acc.at[pl.ds(base, NPT)], out_hbm.at[c, pl.ds(base, NPT)])


@functools.partial(
    pl.kernel,
    out_type=jax.ShapeDtypeStruct((2, NPAD, 128), jnp.float32),
    mesh=_MESH,
    scratch_types=[
        pltpu.VMEM((NCH, CH), jnp.int32),
        pltpu.VMEM((CH, 128), jnp.float32),
        pltpu.VMEM_SHARED((NPAD, 128), jnp.float32),
    ],
)
def _deg_sc(rows_hbm, ones_hbm, zeros_hbm, out_hbm, idx_v, ones_v, acc):
    c = lax.axis_index("c")
    s = lax.axis_index("s")
    w = s * 2 + c
    base = s * NPT
    pltpu.sync_copy(ones_hbm, ones_v)
    pltpu.sync_copy(zeros_hbm.at[pl.ds(base, NPT)], acc.at[pl.ds(base, NPT)])
    plsc.subcore_barrier()
    pltpu.sync_copy(rows_hbm.at[w], idx_v)

    def step(j, carry):
        pltpu.sync_copy(ones_v, acc.at[idx_v.at[j]], add=True)
        return carry

    lax.fori_loop(0, NCH, step, 0)
    plsc.subcore_barrier()
    pltpu.sync_copy(acc.at[pl.ds(base, NPT)], out_hbm.at[c, pl.ds(base, NPT)])


def _dis_body(degp_ref, dis_ref):
    deg = degp_ref[0] + degp_ref[1]
    dis = jnp.where(deg > 0, lax.rsqrt(jnp.maximum(deg, 1e-12)), 0.0)
    dis_ref[...] = dis[:, 0:1]


_dis_tc = pl.pallas_call(
    _dis_body,
    grid=(NPAD // BN + 1,),
    in_specs=[pl.BlockSpec((2, BN, 128), lambda i: (0, i, 0))],
    out_specs=pl.BlockSpec((BN, 1), lambda i: (i, 0)),
    out_shape=jax.ShapeDtypeStruct((NPAD, 1), jnp.float32),
)


def _pre_body(x_ref, w_ref, dis_ref, g1_ref, g2_ref, r_ref):
    p = jnp.dot(x_ref[...], w_ref[...], preferred_element_type=jnp.float32)
    dis = dis_ref[...]
    g1_ref[...] = dis * p[:, 256:384]
    g2_ref[...] = dis * p[:, 128:256]
    r_ref[...] = p[:, 0:128] - p[:, 256:384]


def _make_pre(cin):
    return pl.pallas_call(
        _pre_body,
        grid=(N // BN,),
        in_specs=[
            pl.BlockSpec((BN, cin), lambda i: (i, 0)),
            pl.BlockSpec((cin, 384), lambda i: (0, 0)),
            pl.BlockSpec((BN, 1), lambda i: (i, 0)),
        ],
        out_specs=[
            pl.BlockSpec((BN, 128), lambda i: (i, 0)),
            pl.BlockSpec((BN, 128), lambda i: (i, 0)),
            pl.BlockSpec((BN, 128), lambda i: (i, 0)),
        ],
        out_shape=[jax.ShapeDtypeStruct((N, 128), jnp.float32)] * 3,
    )


_pre_tc = _make_pre(256)


def _mid_body(g2_ref, ap_ref, dis_ref, q_ref):
    dis = dis_ref[...]
    q_ref[...] = g2_ref[...] - 2.0 * dis * dis * (ap_ref[0] + ap_ref[1])


_mid_tc = pl.pallas_call(
    _mid_body,
    grid=(N // BN,),
    in_specs=[
        pl.BlockSpec((BN, 128), lambda i: (i, 0)),
        pl.BlockSpec((2, BN, 128), lambda i: (0, i, 0)),
        pl.BlockSpec((BN, 1), lambda i: (i, 0)),
    ],
    out_specs=pl.BlockSpec((BN, 128), lambda i: (i, 0)),
    out_shape=jax.ShapeDtypeStruct((N, 128), jnp.float32),
)


def _postpre_body(r_ref, ap_ref, dis_ref, b_ref, al_ref, w_ref,
                  g1_ref, g2_ref, r2_ref):
    dis = dis_ref[...]
    out1 = r_ref[...] - dis * (ap_ref[0] + ap_ref[1]) + b_ref[...]
    x2 = jnp.where(out1 >= 0, out1, al_ref[...] * out1)
    p = jnp.dot(x2, w_ref[...], preferred_element_type=jnp.float32)
    g1_ref[...] = dis * p[:, 256:384]
    g2_ref[...] = dis * p[:, 128:256]
    r2_ref[...] = p[:, 0:128] - p[:, 256:384]


_postpre_tc = pl.pallas_call(
    _postpre_body,
    grid=(N // BN,),
    in_specs=[
        pl.BlockSpec((BN, 128), lambda i: (i, 0)),
        pl.BlockSpec((2, BN, 128), lambda i: (0, i, 0)),
        pl.BlockSpec((BN, 1), lambda i: (i, 0)),
        pl.BlockSpec((1, 128), lambda i: (0, 0)),
        pl.BlockSpec((1, 128), lambda i: (0, 0)),
        pl.BlockSpec((128, 384), lambda i: (0, 0)),
    ],
    out_specs=[
        pl.BlockSpec((BN, 128), lambda i: (i, 0)),
        pl.BlockSpec((BN, 128), lambda i: (i, 0)),
        pl.BlockSpec((BN, 128), lambda i: (i, 0)),
    ],
    out_shape=[jax.ShapeDtypeStruct((N, 128), jnp.float32)] * 3,
)


def _pool_body(r_ref, ap_ref, dis_ref, b_ref, al_ref, batch_ref,
               z_ref, zsum, csum):
    i = pl.program_id(0)

    @pl.when(i == 0)
    def _init():
        zsum[...] = jnp.zeros_like(zsum)
        csum[...] = jnp.zeros_like(csum)

    dis = dis_ref[...]
    out2 = r_ref[...] - dis * (ap_ref[0] + ap_ref[1]) + b_ref[...]
    h = jnp.where(out2 >= 0, out2, al_ref[...] * out2)
    gids = lax.broadcasted_iota(jnp.int32, (BN, NG), 1)
    mask = (batch_ref[...] == gids).astype(jnp.float32)
    dn = (((0,), (0,)), ((), ()))
    zsum[...] += lax.dot_general(mask, h, dn,
                                 preferred_element_type=jnp.float32)
    csum[...] += lax.dot_general(mask, jnp.ones_like(h), dn,
                                 preferred_element_type=jnp.float32)

    @pl.when(i == N // BN - 1)
    def _fin():
        z_ref[...] = zsum[...] / jnp.maximum(csum[...], 1.0)


_pool_tc = pl.pallas_call(
    _pool_body,
    grid=(N // BN,),
    in_specs=[
        pl.BlockSpec((BN, 128), lambda i: (i, 0)),
        pl.BlockSpec((2, BN, 128), lambda i: (0, i, 0)),
        pl.BlockSpec((BN, 1), lambda i: (i, 0)),
        pl.BlockSpec((1, 128), lambda i: (0, 0)),
        pl.BlockSpec((1, 128), lambda i: (0, 0)),
        pl.BlockSpec((BN, 1), lambda i: (i, 0)),
    ],
    out_specs=pl.BlockSpec((NG, 128), lambda i: (0, 0)),
    out_shape=jax.ShapeDtypeStruct((NG, 128), jnp.float32),
    scratch_shapes=[
        pltpu.VMEM((NG, 128), jnp.float32),
        pltpu.VMEM((NG, 128), jnp.float32),
    ],
)


def _cls_body(za_ref, zh_ref, w1_ref, b1_ref, a_ref, w2_ref, b2_ref,
              logits_ref, z_ref):
    z = jnp.concatenate([za_ref[...], zh_ref[...]], axis=1)
    h0 = jnp.dot(z, w1_ref[...], preferred_element_type=jnp.float32) + b1_ref[...]
    h = jnp.where(h0 >= 0, h0, a_ref[...] * h0)
    logits_ref[...] = (jnp.dot(h, w2_ref[...],
                               preferred_element_type=jnp.float32)
                       + b2_ref[...])
    z_ref[...] = z


_cls_tc = pl.pallas_call(
    _cls_body,
    out_shape=[
        jax.ShapeDtypeStruct((NG, 2), jnp.float32),
        jax.ShapeDtypeStruct((NG, 256), jnp.float32),
    ],
)


def kernel(x, edge_index_asd, edge_index_hc, batch,
           asd_W1, asd_b1, asd_a1, asd_W2, asd_b2, asd_a2,
           hc_W1, hc_b1, hc_a1, hc_W2, hc_b2, hc_a2,
           cls_W1, cls_b1, cls_a, cls_W2, cls_b2):
    zeros128 = jnp.zeros((NPAD, 128), jnp.float32)
    ones128 = jnp.ones((CH, 128), jnp.float32)

    rows_a = edge_index_asd[0].reshape(NW, NCH, CH)
    rows_h = edge_index_hc[0].reshape(NW, NCH, CH)
    degp_a = _deg_sc(rows_a, ones128, zeros128)
    degp_h = _deg_sc(rows_h, ones128, zeros128)
    dis_a = _dis_tc(degp_a)[:N]
    dis_h = _dis_tc(degp_h)[:N]
    batch2 = batch.reshape(N, 1)

    def run_branch(ei, dis, W1, b1, a1, W2, b2, a2):
        rows = ei[0].reshape(NW, NCH, CH)
        cols = ei[1].reshape(NW, NCH, CH)
        wc1 = jnp.concatenate([W1[0], W1[1], W1[2]], axis=1)
        wc2 = jnp.concatenate([W2[0], W2[1], W2[2]], axis=1)
        g1, g2, r = _pre_tc(x, wc1, dis)
        ap1 = _prop_sc(g1, rows, cols, zeros128)
        q = _mid_tc(g2, ap1, dis)
        ap2 = _prop_sc(q, rows, cols, zeros128)
        g1b, g2b, rb = _postpre_tc(r, ap2, dis, b1.reshape(1, 128),
                                   a1.reshape(1, 128), wc2)
        ap3 = _prop_sc(g1b, rows, cols, zeros128)
        qb = _mid_tc(g2b, ap3, dis)
        ap4 = _prop_sc(qb, rows, cols, zeros128)
        return _pool_tc(rb, ap4, dis, b2.reshape(1, 128),
                        a2.reshape(1, 128), batch2)

    z_a = run_branch(edge_index_asd, dis_a,
                     asd_W1, asd_b1, asd_a1, asd_W2, asd_b2, asd_a2)
    z_h = run_branch(edge_index_hc, dis_h,
                     hc_W1, hc_b1, hc_a1, hc_W2, hc_b2, hc_a2)
    logits, z = _cls_tc(z_a, z_h, cls_W1, cls_b1.reshape(1, 256),
                        cls_a.reshape(1, 256), cls_W2, cls_b2.reshape(1, 2))
    return logits, z
